# baseline (speedup 1.0000x reference)
import functools

import jax
import jax.numpy as jnp
from jax import lax
from jax.experimental import pallas as pl
from jax.experimental.pallas import tpu as pltpu
from jax.experimental.pallas import tpu_sc as plsc

B, T = 4, 8192
V, D = 8192, 1024
B_TOTAL = B * T
NC, NS = 2, 16
NW = NC * NS
B_PER_W = B_TOTAL // NW
W_PER_ROW = T // B_PER_W
CHUNK = 8
N_CHUNKS = B_PER_W // CHUNK
NBUF = 8


def _sc_gather(t_indices, pe):
    mesh = plsc.VectorSubcoreMesh(core_axis_name="c", subcore_axis_name="s")

    @functools.partial(
        pl.kernel,
        mesh=mesh,
        out_type=jax.ShapeDtypeStruct((B_TOTAL, D), jnp.float32),
        scratch_types=[
            pltpu.VMEM((B_PER_W,), jnp.int32),
            [pltpu.VMEM((CHUNK, D), jnp.float32)] * NBUF,
            [pltpu.SemaphoreType.DMA] * NBUF,
            [pltpu.SemaphoreType.DMA] * NBUF,
        ],
    )
    def k(idx_hbm, table_hbm, out_hbm, idx_v, bufs, gsems, wsems):
        wid = lax.axis_index("s") * NC + lax.axis_index("c")
        base = wid * B_PER_W
        pltpu.sync_copy(
            idx_hbm.at[wid // W_PER_ROW, pl.ds((wid % W_PER_ROW) * B_PER_W, B_PER_W)],
            idx_v,
        )

        def gather_start(c, b):
            pltpu.async_copy(
                table_hbm.at[idx_v.at[pl.ds(c * CHUNK, CHUNK)]], bufs[b], gsems[b]
            )

        def gather_wait(b):
            pltpu.make_async_copy(
                table_hbm.at[idx_v.at[pl.ds(0, CHUNK)]], bufs[b], gsems[b]
            ).wait()

        def write_start(c, b):
            pltpu.async_copy(
                bufs[b], out_hbm.at[pl.ds(base + c * CHUNK, CHUNK)], wsems[b]
            )

        def write_wait(b):
            pltpu.make_async_copy(
                bufs[b], out_hbm.at[pl.ds(base, CHUNK)], wsems[b]
            ).wait()

        for b in range(NBUF):
            gather_start(b, b)

        @pl.loop(0, N_CHUNKS - NBUF, step=NBUF)
        def group(g):
            for b in range(NBUF):
                gather_wait(b)
                write_start(g + b, b)
            for b in range(NBUF):
                write_wait(b)
                gather_start(g + NBUF + b, b)

        for b in range(NBUF):
            gather_wait(b)
            write_start(N_CHUNKS - NBUF + b, b)
        for b in range(NBUF):
            write_wait(b)

    return k(t_indices, pe)


@jax.jit
def kernel(t_indices, pe):
    out = _sc_gather(t_indices.astype(jnp.int32), pe)
    return out.reshape(B, T, D)


# speedup vs baseline: 1.0252x; 1.0252x over previous
import functools

import jax
import jax.numpy as jnp
from jax import lax
from jax.experimental import pallas as pl
from jax.experimental.pallas import tpu as pltpu
from jax.experimental.pallas import tpu_sc as plsc

B, T = 4, 8192
V, D = 8192, 1024
B_TOTAL = B * T
NC, NS = 2, 16
NW = NC * NS
B_PER_W = B_TOTAL // NW
W_PER_ROW = T // B_PER_W
CHUNK = 8
N_CHUNKS = B_PER_W // CHUNK
NBUF = 8


def _sc_gather(t_indices, pe):
    mesh = plsc.VectorSubcoreMesh(core_axis_name="c", subcore_axis_name="s")

    @functools.partial(
        pl.kernel,
        mesh=mesh,
        out_type=jax.ShapeDtypeStruct((B_TOTAL, D), jnp.float32),
        scratch_types=[
            pltpu.VMEM((B_PER_W,), jnp.int32),
            [pltpu.VMEM((CHUNK, D), jnp.float32)] * NBUF,
            [pltpu.SemaphoreType.DMA] * NBUF,
            [pltpu.SemaphoreType.DMA] * NBUF,
        ],
    )
    def k(idx_hbm, table_hbm, out_hbm, idx_v, bufs, gsems, wsems):
        wid = lax.axis_index("s") * NC + lax.axis_index("c")
        base = wid * B_PER_W
        pltpu.sync_copy(
            idx_hbm.at[wid // W_PER_ROW, pl.ds((wid % W_PER_ROW) * B_PER_W, B_PER_W)],
            idx_v,
        )

        def gather_start(c, b):
            pltpu.async_copy(
                table_hbm.at[idx_v.at[pl.ds(c * CHUNK, CHUNK)]], bufs[b], gsems[b]
            )

        def gather_wait(b):
            pltpu.make_async_copy(
                table_hbm.at[idx_v.at[pl.ds(0, CHUNK)]], bufs[b], gsems[b]
            ).wait()

        def write_start(c, b):
            pltpu.async_copy(
                bufs[b], out_hbm.at[pl.ds(base + c * CHUNK, CHUNK)], wsems[b]
            )

        def write_wait(b):
            pltpu.make_async_copy(
                bufs[b], out_hbm.at[pl.ds(base, CHUNK)], wsems[b]
            ).wait()

        for b in range(NBUF):
            gather_start(b, b)

        @pl.loop(0, N_CHUNKS - NBUF, step=NBUF)
        def group(g):
            for b in range(NBUF):
                gather_wait(b)
                write_start(g + b, b)
                write_wait(b)
                gather_start(g + NBUF + b, b)

        for b in range(NBUF):
            gather_wait(b)
            write_start(N_CHUNKS - NBUF + b, b)
        for b in range(NBUF):
            write_wait(b)

    return k(t_indices, pe)


@jax.jit
def kernel(t_indices, pe):
    out = _sc_gather(t_indices.astype(jnp.int32), pe)
    return out.reshape(B, T, D)
